# Initial kernel scaffold; baseline (speedup 1.0000x reference)
import jax
import jax.numpy as jnp
from jax.experimental import pallas as pl


def kernel(input_spatial, trunk_out, row_embed, col_embed, W1, b1, W2, b2):
    raise NotImplementedError("write your pallas kernel here")


# speedup vs baseline: 3.3289x; 3.3289x over previous
import functools

import jax
import jax.numpy as jnp
from jax import lax
from jax.experimental import pallas as pl
from jax.experimental.pallas import tpu as pltpu
from jax.experimental.pallas import tpu_sc as plsc

_POS_LEN = 19
_NUM_HIST = 5
_HW = 361
_EMBED_PER = 64
_BN = 512


def _argmax_body(x_ref, comb_ref):
    flat = x_ref[...]
    s = jnp.sum(flat, axis=-1)
    m = jnp.max(flat, axis=-1)
    iota = lax.broadcasted_iota(jnp.int32, flat.shape, 2)
    idx = jnp.min(jnp.where(flat == m[..., None], iota, jnp.int32(1 << 20)),
                  axis=-1)
    has = s > 0.5
    rows = (idx * 27) >> 9
    cols = idx - rows * 19
    comb_ref[...] = jnp.where(has, rows * (_POS_LEN + 1) + cols,
                              (_POS_LEN + 1) * (_POS_LEN + 1) - 1)


def _extract_indices(x5):
    n = x5.shape[0]
    return pl.pallas_call(
        _argmax_body,
        grid=(n // _BN,),
        in_specs=[pl.BlockSpec((_BN, _NUM_HIST, _HW), lambda i: (i, 0, 0))],
        out_specs=pl.BlockSpec((_BN, _NUM_HIST), lambda i: (i, 0)),
        out_shape=jax.ShapeDtypeStruct((n, _NUM_HIST), jnp.int32),
    )(x5)


def _sc_gather(table, idx):
    info = plsc.get_sparse_core_info()
    nw = info.num_cores * info.num_subcores
    n = idx.shape[0]
    rpw = n // nw
    cpw = rpw // 128
    d = table.shape[1]
    mesh = plsc.VectorSubcoreMesh(core_axis_name="c", subcore_axis_name="s")

    @functools.partial(
        pl.kernel,
        mesh=mesh,
        out_type=jax.ShapeDtypeStruct((n, d), jnp.float32),
        scratch_types=[
            pltpu.VMEM((rpw,), jnp.int32),
            pltpu.VMEM((rpw, d), jnp.float32),
            pltpu.SemaphoreType.DMA,
        ],
    )
    def gather_kernel(table_hbm, idx_hbm, out_hbm, idx_v, rows_v, sem):
        wid = lax.axis_index("s") * info.num_cores + lax.axis_index("c")
        pltpu.sync_copy(idx_hbm.at[pl.ds(wid * rpw, rpw)], idx_v)
        copies = [
            pltpu.async_copy(table_hbm.at[idx_v.at[pl.ds(j * 128, 128)]],
                             rows_v.at[pl.ds(j * 128, 128)], sem)
            for j in range(cpw)
        ]
        for c in copies:
            c.wait()
        pltpu.sync_copy(rows_v, out_hbm.at[pl.ds(wid * rpw, rpw)])

    return gather_kernel(table, idx)


def _mlp_body(e_ref, w1_ref, b1_ref, w2_ref, b2_ref, o_ref):
    h = jnp.dot(e_ref[...], w1_ref[...], preferred_element_type=jnp.float32)
    h = jnp.maximum(h + b1_ref[...], 0.0)
    o_ref[...] = (
        jnp.dot(h, w2_ref[...], preferred_element_type=jnp.float32)
        + b2_ref[...]
    )


def _mlp(e, w1t, b1, w2t, b2):
    n, fan_in = e.shape
    hidden = w1t.shape[1]
    c_out = w2t.shape[1]
    return pl.pallas_call(
        _mlp_body,
        grid=(n // _BN,),
        in_specs=[
            pl.BlockSpec((_BN, fan_in), lambda i: (i, 0)),
            pl.BlockSpec((fan_in, hidden), lambda i: (0, 0)),
            pl.BlockSpec((1, hidden), lambda i: (0, 0)),
            pl.BlockSpec((hidden, c_out), lambda i: (0, 0)),
            pl.BlockSpec((1, c_out), lambda i: (0, 0)),
        ],
        out_specs=pl.BlockSpec((_BN, c_out), lambda i: (i, 0)),
        out_shape=jax.ShapeDtypeStruct((n, c_out), jnp.float32),
    )(e, w1t, b1, w2t, b2)


def kernel(input_spatial, trunk_out, row_embed, col_embed, W1, b1, W2, b2):
    n = input_spatial.shape[0]
    x5 = input_spatial[:, 9:14, :, :].reshape(n, _NUM_HIST, _HW)
    comb = _extract_indices(x5).reshape(-1)
    table = jnp.concatenate(
        [jnp.repeat(row_embed, _POS_LEN + 1, axis=0),
         jnp.tile(col_embed, (_POS_LEN + 1, 1))], axis=-1)
    emb = _sc_gather(table, comb)
    out = _mlp(
        emb.reshape(n, 2 * _NUM_HIST * _EMBED_PER),
        W1.T, b1.reshape(1, -1), W2.T, b2.reshape(1, -1),
    )
    return out[:, :, None, None]
